# baseline (speedup 1.0000x reference)
import functools

import jax
import jax.numpy as jnp
from jax.experimental import pallas as pl
from jax.experimental.pallas import tpu as pltpu
from jax.experimental.pallas import tpu_sc as plsc

_K = 8192
_D = 256
_BN = 512
_COMMIT = 0.25

_NC = 2
_NS = 16
_NW = _NC * _NS
_CH = 128


def _gather_body(w_hbm, idx_hbm, out_hbm, idx_v, rows_v, sem):
    nch = idx_v.shape[0]
    ch_tokens = _CH
    wid = jax.lax.axis_index("s") * _NC + jax.lax.axis_index("c")
    pltpu.sync_copy(idx_hbm.at[wid], idx_v)
    cps = [
        pltpu.async_copy(w_hbm.at[idx_v.at[j]], rows_v.at[j], sem)
        for j in range(nch)
    ]
    for j, cp in enumerate(cps):
        cp.wait()
        pltpu.sync_copy(
            rows_v.at[j],
            out_hbm.at[pl.ds((wid * nch + j) * ch_tokens, ch_tokens)])


def _sc_gather(weight, idx):
    n = idx.shape[0]
    nch = n // (_NW * _CH)
    mesh = plsc.VectorSubcoreMesh(
        core_axis_name="c", subcore_axis_name="s",
        num_cores=_NC, num_subcores=_NS)
    run = pl.kernel(
        _gather_body,
        out_type=jax.ShapeDtypeStruct((n, _D), jnp.float32),
        mesh=mesh,
        scratch_types=[
            pltpu.VMEM((nch, _CH), jnp.int32),
            pltpu.VMEM((nch, _CH, _D), jnp.float32),
            pltpu.SemaphoreType.DMA,
        ],
    )
    return run(weight, idx.reshape(_NW, nch, _CH))


_CW = 256


def _argmin_body(z_ref, w_ref, a_ref, b_ref, idx_ref, acc_ref):
    i = pl.program_id(0)
    bn = z_ref.shape[0]
    zm2 = z_ref[...] * jnp.float32(-2.0)
    a = a_ref[...]

    m_acc = jnp.full((bn, _CW), jnp.inf, jnp.float32)
    i_acc = jnp.zeros((bn, _CW), jnp.int32)
    for c in range(_K // _CW):
        w_c = w_ref[c * _CW:(c + 1) * _CW, :]
        mm2 = jax.lax.dot_general(
            zm2, w_c, (((1,), (1,)), ((), ())),
            preferred_element_type=jnp.float32)
        d = (a + b_ref[:, c * _CW:(c + 1) * _CW]) + mm2
        upd = d < m_acc
        i_acc = jnp.where(upd, jnp.int32(c), i_acc)
        m_acc = jnp.minimum(m_acc, d)

    m = jnp.min(m_acc, axis=1, keepdims=True)
    col = jax.lax.broadcasted_iota(jnp.int32, (bn, _CW), 1)
    key = jnp.where(m_acc == m, i_acc * _CW + col, jnp.int32(2**30))
    idx_ref[0, 0, :] = jnp.min(key, axis=1)

    @pl.when(i == 0)
    def _():
        acc_ref[...] = jnp.zeros((1, 1), jnp.float32)

    acc_ref[...] += jnp.sum(m).reshape(1, 1)


def kernel(z, weight):
    B, T, D = z.shape
    N = B * T
    flat = z.reshape(N, D)
    nb = N // _BN

    a = jnp.sum(flat ** 2, axis=1, keepdims=True)
    b = jnp.sum(weight ** 2, axis=1).reshape(1, _K)

    def _argmin_half(flat_h, a_h):
        nbh = flat_h.shape[0] // _BN
        return pl.pallas_call(
            _argmin_body,
            grid=(nbh,),
            in_specs=[
                pl.BlockSpec((_BN, D), lambda i: (i, 0)),
                pl.BlockSpec((_K, D), lambda i: (0, 0)),
                pl.BlockSpec((_BN, 1), lambda i: (i, 0)),
                pl.BlockSpec((1, _K), lambda i: (0, 0)),
            ],
            out_specs=[
                pl.BlockSpec((1, 1, _BN), lambda i: (i, 0, 0)),
                pl.BlockSpec((1, 1), lambda i: (0, 0)),
            ],
            out_shape=[
                jax.ShapeDtypeStruct((nbh, 1, _BN), jnp.int32),
                jax.ShapeDtypeStruct((1, 1), jnp.float32),
            ],
        )(flat_h, weight, a_h, b)

    h = N // 2
    idx3_0, acc0 = _argmin_half(flat[:h], a[:h])
    q0 = _sc_gather(weight, idx3_0.reshape(h))
    idx3_1, acc1 = _argmin_half(flat[h:], a[h:])
    q1 = _sc_gather(weight, idx3_1.reshape(h))

    idx = jnp.concatenate([idx3_0.reshape(h), idx3_1.reshape(h)])
    loss = ((acc0[0, 0] + acc1[0, 0]) / (N * D)) * (1.0 + _COMMIT)

    quantized = jnp.concatenate([q0, q1]).reshape(B, T, D)
    quantized_st = z + (quantized - z)
    return quantized_st, loss, idx.reshape(B, T)


# speedup vs baseline: 1.2469x; 1.2469x over previous
import functools

import jax
import jax.numpy as jnp
from jax.experimental import pallas as pl
from jax.experimental.pallas import tpu as pltpu
from jax.experimental.pallas import tpu_sc as plsc

_K = 8192
_D = 256
_BN = 512
_COMMIT = 0.25

_NC = 2
_NS = 16
_NW = _NC * _NS
_CH = 128


def _gather_body(w_hbm, idx_hbm, out_hbm, idx_v, rows_v, sem):
    nch = idx_v.shape[0]
    ch_tokens = _CH
    wid = jax.lax.axis_index("s") * _NC + jax.lax.axis_index("c")
    pltpu.sync_copy(idx_hbm.at[wid], idx_v)
    cps = [
        pltpu.async_copy(w_hbm.at[idx_v.at[j]], rows_v.at[j], sem)
        for j in range(nch)
    ]
    for j, cp in enumerate(cps):
        cp.wait()
        pltpu.sync_copy(
            rows_v.at[j],
            out_hbm.at[pl.ds((wid * nch + j) * ch_tokens, ch_tokens)])


def _sc_gather(weight, idx):
    n = idx.shape[0]
    nch = n // (_NW * _CH)
    mesh = plsc.VectorSubcoreMesh(
        core_axis_name="c", subcore_axis_name="s",
        num_cores=_NC, num_subcores=_NS)
    run = pl.kernel(
        _gather_body,
        out_type=jax.ShapeDtypeStruct((n, _D), jnp.float32),
        mesh=mesh,
        scratch_types=[
            pltpu.VMEM((nch, _CH), jnp.int32),
            pltpu.VMEM((nch, _CH, _D), jnp.float32),
            pltpu.SemaphoreType.DMA,
        ],
    )
    return run(weight, idx.reshape(_NW, nch, _CH))


_CW = 256


def _argmin_body(z_ref, w_ref, a_ref, b_ref, idx_ref, acc_ref):
    i = pl.program_id(0)
    bn = z_ref.shape[0]
    zm2 = z_ref[...] * jnp.float32(-2.0)
    a = a_ref[...]

    m_acc = jnp.full((bn, _CW), jnp.inf, jnp.float32)
    i_acc = jnp.zeros((bn, _CW), jnp.int32)
    for c in range(_K // _CW):
        w_c = w_ref[c * _CW:(c + 1) * _CW, :]
        mm2 = jax.lax.dot_general(
            zm2, w_c, (((1,), (1,)), ((), ())),
            preferred_element_type=jnp.float32)
        d = (a + b_ref[:, c * _CW:(c + 1) * _CW]) + mm2
        upd = d < m_acc
        i_acc = jnp.where(upd, jnp.int32(c), i_acc)
        m_acc = jnp.minimum(m_acc, d)

    m = jnp.min(m_acc, axis=1, keepdims=True)
    col = jax.lax.broadcasted_iota(jnp.int32, (bn, _CW), 1)
    key = jnp.where(m_acc == m, i_acc * _CW + col, jnp.int32(2**30))
    idx_ref[0, 0, :] = jnp.min(key, axis=1)

    @pl.when(i == 0)
    def _():
        acc_ref[...] = jnp.zeros((1, 1), jnp.float32)

    acc_ref[...] += jnp.sum(m).reshape(1, 1)


def kernel(z, weight):
    B, T, D = z.shape
    N = B * T
    flat = z.reshape(N, D)
    nb = N // _BN

    a = jnp.sum(flat ** 2, axis=1, keepdims=True)
    b = jnp.sum(weight ** 2, axis=1).reshape(1, _K)

    idx3, acc = pl.pallas_call(
        _argmin_body,
        grid=(nb,),
        in_specs=[
            pl.BlockSpec((_BN, D), lambda i: (i, 0)),
            pl.BlockSpec((_K, D), lambda i: (0, 0)),
            pl.BlockSpec((_BN, 1), lambda i: (i, 0)),
            pl.BlockSpec((1, _K), lambda i: (0, 0)),
        ],
        out_specs=[
            pl.BlockSpec((1, 1, _BN), lambda i: (i, 0, 0)),
            pl.BlockSpec((1, 1), lambda i: (0, 0)),
        ],
        out_shape=[
            jax.ShapeDtypeStruct((nb, 1, _BN), jnp.int32),
            jax.ShapeDtypeStruct((1, 1), jnp.float32),
        ],
    )(flat, weight, a, b)

    idx = idx3.reshape(N)
    loss = (acc[0, 0] / (N * D)) * (1.0 + _COMMIT)

    quantized_st = _sc_gather(weight, idx).reshape(B, T, D)
    return quantized_st, loss, idx.reshape(B, T)
